# baseline (speedup 1.0000x reference)
import functools
import jax
import jax.numpy as jnp
from jax import lax
from jax.experimental import pallas as pl
from jax.experimental.pallas import tpu as pltpu
from jax.experimental.pallas import tpu_sc as plsc

B, N, K, D = 1024, 500, 50, 128
R = 2 + K
RPAD = 64
NW = 32
BPW = B // NW
L = 16


def _sc_gather(nodes_hbm, idx_hbm, gx_hbm, gy_hbm,
               nodes_v, idx_v, gx_v, gy_v):
    wid = lax.axis_index("s") * 2 + lax.axis_index("c")
    b0 = wid * BPW
    pltpu.sync_copy(nodes_hbm.at[pl.ds(b0 * N * 2, BPW * N * 2)], nodes_v)
    pltpu.sync_copy(idx_hbm.at[pl.ds(b0 * RPAD, BPW * RPAD)], idx_v)

    def body(b, _):
        base = b * N * 2
        for c in range(RPAD // L):
            ids = idx_v[pl.ds(b * RPAD + c * L, L)]
            fi = ids * 2 + base
            gx_v[pl.ds(b * RPAD + c * L, L)] = plsc.load_gather(
                nodes_v, [fi])
            gy_v[pl.ds(b * RPAD + c * L, L)] = plsc.load_gather(
                nodes_v, [fi + 1])
        return 0

    lax.fori_loop(0, BPW, body, 0)
    pltpu.sync_copy(gx_v, gx_hbm.at[pl.ds(b0 * RPAD, BPW * RPAD)])
    pltpu.sync_copy(gy_v, gy_hbm.at[pl.ds(b0 * RPAD, BPW * RPAD)])


_sc_call = pl.kernel(
    _sc_gather,
    out_type=(
        jax.ShapeDtypeStruct((B * RPAD,), jnp.float32),
        jax.ShapeDtypeStruct((B * RPAD,), jnp.float32),
    ),
    mesh=plsc.VectorSubcoreMesh(core_axis_name="c", subcore_axis_name="s"),
    compiler_params=pltpu.CompilerParams(needs_layout_passes=False),
    scratch_types=[
        pltpu.VMEM((BPW * N * 2,), jnp.float32),
        pltpu.VMEM((BPW * RPAD,), jnp.int32),
        pltpu.VMEM((BPW * RPAD,), jnp.float32),
        pltpu.VMEM((BPW * RPAD,), jnp.float32),
    ],
)


TB = 8


def _tc_dense(gx_ref, gy_ref, wrows_ref, w1_ref, w2_ref, out_ref):
    gx = gx_ref[...]
    gy = gy_ref[...]
    wx = wrows_ref[0, :]
    wy = wrows_ref[1, :]
    bc = wrows_ref[2, :]
    w1b = wrows_ref[3, :]
    w2b = wrows_ref[4, :]
    emb = gx[:, :, None] * wx[None, None, :] \
        + gy[:, :, None] * wy[None, None, :] + bc[None, None, :]
    f = jnp.dot(emb[:, 0, :], w1_ref[...],
                preferred_element_type=jnp.float32) + w1b[None, :]
    l = jnp.dot(emb[:, 1, :], w2_ref[...],
                preferred_element_type=jnp.float32) + w2b[None, :]
    out_ref[...] = jnp.concatenate(
        [f[:, None, :], l[:, None, :], emb[:, 2:R, :]], axis=1)


def kernel(nodes, first_node_idx, last_node_idx, candidate_indices,
           W_coord, b_coord, W1_w, W1_b, W2_w, W2_b):
    idx = jnp.concatenate(
        [first_node_idx[:, None], last_node_idx[:, None],
         jnp.clip(candidate_indices, 0, None)], axis=1).astype(jnp.int32)
    idx = jnp.pad(idx, ((0, 0), (0, RPAD - R)))
    wrows = jnp.zeros((8, D), jnp.float32)
    wrows = wrows.at[0].set(W_coord[0]).at[1].set(W_coord[1])
    wrows = wrows.at[2].set(b_coord).at[3].set(W1_b).at[4].set(W2_b)

    gx, gy = _sc_call(nodes.reshape(B * N * 2), idx.reshape(B * RPAD))
    gx = gx.reshape(B, RPAD)
    gy = gy.reshape(B, RPAD)

    grid = (B // TB,)
    out = pl.pallas_call(
        _tc_dense,
        grid=grid,
        in_specs=[
            pl.BlockSpec((TB, RPAD), lambda i: (i, 0)),
            pl.BlockSpec((TB, RPAD), lambda i: (i, 0)),
            pl.BlockSpec((8, D), lambda i: (0, 0)),
            pl.BlockSpec((D, D), lambda i: (0, 0)),
            pl.BlockSpec((D, D), lambda i: (0, 0)),
        ],
        out_specs=pl.BlockSpec((TB, R, D), lambda i: (i, 0, 0)),
        out_shape=jax.ShapeDtypeStruct((B, R, D), jnp.float32),
    )(gx, gy, wrows, W1_w, W2_w)
    return out


# speedup vs baseline: 3.1140x; 3.1140x over previous
import functools
import jax
import jax.numpy as jnp
from jax import lax
from jax.experimental import pallas as pl
from jax.experimental.pallas import tpu as pltpu
from jax.experimental.pallas import tpu_sc as plsc

B, N, K, D = 1024, 500, 50, 128
R = 2 + K
RPAD = 64
NW = 32
BPW = B // NW
L = 16


NG = B // 128
BCH = 16


def _sc_gather(nodes_hbm, idx_hbm, gx_hbm, gy_hbm,
               nodes_v, idx_v, g_v):
    wid = lax.axis_index("s") * 2 + lax.axis_index("c")

    @pl.when(wid < NG)
    def _():
        b0 = wid * 128
        pltpu.sync_copy(nodes_hbm.at[:, :, pl.ds(b0, 128)], nodes_v)
        zeros = jnp.zeros((L,), jnp.int32)
        ones = jnp.ones((L,), jnp.int32)

        def chunk(k, _):
            pltpu.sync_copy(
                idx_hbm.at[pl.ds((b0 + k * BCH) * RPAD, BCH * RPAD)], idx_v)

            def gather_plane(coord_vec, out_hbm):
                def body(b, _):
                    bvec = jnp.full((L,), k * BCH + b, jnp.int32)
                    for c in range(RPAD // L):
                        ids = idx_v[pl.ds(b * RPAD + c * L, L)]
                        g_v[pl.ds(b * RPAD + c * L, L)] = plsc.load_gather(
                            nodes_v, [ids, coord_vec, bvec])
                    return 0
                lax.fori_loop(0, BCH, body, 0)
                pltpu.sync_copy(
                    g_v, out_hbm.at[pl.ds((b0 + k * BCH) * RPAD, BCH * RPAD)])

            gather_plane(zeros, gx_hbm)
            gather_plane(ones, gy_hbm)
            return 0

        lax.fori_loop(0, 128 // BCH, chunk, 0)


_sc_call = pl.kernel(
    _sc_gather,
    out_type=(
        jax.ShapeDtypeStruct((B * RPAD,), jnp.float32),
        jax.ShapeDtypeStruct((B * RPAD,), jnp.float32),
    ),
    mesh=plsc.VectorSubcoreMesh(core_axis_name="c", subcore_axis_name="s"),
    compiler_params=pltpu.CompilerParams(needs_layout_passes=False),
    scratch_types=[
        pltpu.VMEM((N, 2, 128), jnp.float32),
        pltpu.VMEM((BCH * RPAD,), jnp.int32),
        pltpu.VMEM((BCH * RPAD,), jnp.float32),
    ],
)


TB = 8


def _tc_dense(gx_ref, gy_ref, wrows_ref, w1_ref, w2_ref, out_ref):
    gx = gx_ref[...]
    gy = gy_ref[...]
    wx = wrows_ref[0, :]
    wy = wrows_ref[1, :]
    bc = wrows_ref[2, :]
    w1b = wrows_ref[3, :]
    w2b = wrows_ref[4, :]
    emb = gx[:, :, None] * wx[None, None, :] \
        + gy[:, :, None] * wy[None, None, :] + bc[None, None, :]
    f = jnp.dot(emb[:, 0, :], w1_ref[...],
                preferred_element_type=jnp.float32) + w1b[None, :]
    l = jnp.dot(emb[:, 1, :], w2_ref[...],
                preferred_element_type=jnp.float32) + w2b[None, :]
    out_ref[...] = jnp.concatenate(
        [f[:, None, :], l[:, None, :], emb[:, 2:R, :]], axis=1)


def kernel(nodes, first_node_idx, last_node_idx, candidate_indices,
           W_coord, b_coord, W1_w, W1_b, W2_w, W2_b):
    idx = jnp.concatenate(
        [first_node_idx[:, None], last_node_idx[:, None],
         jnp.clip(candidate_indices, 0, None)], axis=1).astype(jnp.int32)
    idx = jnp.pad(idx, ((0, 0), (0, RPAD - R)))
    wrows = jnp.zeros((8, D), jnp.float32)
    wrows = wrows.at[0].set(W_coord[0]).at[1].set(W_coord[1])
    wrows = wrows.at[2].set(b_coord).at[3].set(W1_b).at[4].set(W2_b)

    nodes_t = jnp.transpose(nodes, (1, 2, 0))
    gx, gy = _sc_call(nodes_t, idx.reshape(B * RPAD))
    gx = gx.reshape(B, RPAD)
    gy = gy.reshape(B, RPAD)

    grid = (B // TB,)
    out = pl.pallas_call(
        _tc_dense,
        grid=grid,
        in_specs=[
            pl.BlockSpec((TB, RPAD), lambda i: (i, 0)),
            pl.BlockSpec((TB, RPAD), lambda i: (i, 0)),
            pl.BlockSpec((8, D), lambda i: (0, 0)),
            pl.BlockSpec((D, D), lambda i: (0, 0)),
            pl.BlockSpec((D, D), lambda i: (0, 0)),
        ],
        out_specs=pl.BlockSpec((TB, R, D), lambda i: (i, 0, 0)),
        out_shape=jax.ShapeDtypeStruct((B, R, D), jnp.float32),
    )(gx, gy, wrows, W1_w, W2_w)
    return out


# speedup vs baseline: 7.8112x; 2.5084x over previous
import jax
import jax.numpy as jnp
from jax import lax
from jax.experimental import pallas as pl
from jax.experimental.pallas import tpu as pltpu
from jax.experimental.pallas import tpu_sc as plsc

B, N, K, D = 1024, 500, 50, 128
R = 2 + K
RPAD = 64
L = 16
NT = 16
GB = 128


def _sc_gather(nodes_hbm, idx_hbm, gx_hbm, gy_hbm, nodes_v, idx_v, g_v):
    wid = lax.axis_index("s") * 2 + lax.axis_index("c")

    @pl.when(wid < NT)
    def _():
        p = wid % 2
        b0 = (wid // 2) * GB
        pltpu.sync_copy(nodes_hbm.at[:, p, pl.ds(b0, GB)], nodes_v)
        pltpu.sync_copy(idx_hbm.at[pl.ds(b0 * RPAD, GB * RPAD)], idx_v)

        def body(b, _):
            bvec = jnp.full((L,), b, jnp.int32)
            for c in range(RPAD // L):
                o = b * RPAD + c * L
                ids = idx_v[pl.ds(o, L)]
                g_v[pl.ds(o, L)] = plsc.load_gather(nodes_v, [ids, bvec])
            return 0

        lax.fori_loop(0, GB, body, 0)

        @pl.when(p == 0)
        def _():
            pltpu.sync_copy(g_v, gx_hbm.at[pl.ds(b0 * RPAD, GB * RPAD)])

        @pl.when(p == 1)
        def _():
            pltpu.sync_copy(g_v, gy_hbm.at[pl.ds(b0 * RPAD, GB * RPAD)])


_sc_call = pl.kernel(
    _sc_gather,
    out_type=(
        jax.ShapeDtypeStruct((B * RPAD,), jnp.float32),
        jax.ShapeDtypeStruct((B * RPAD,), jnp.float32),
    ),
    mesh=plsc.VectorSubcoreMesh(core_axis_name="c", subcore_axis_name="s"),
    compiler_params=pltpu.CompilerParams(needs_layout_passes=False),
    scratch_types=[
        pltpu.VMEM((N, GB), jnp.float32),
        pltpu.VMEM((GB * RPAD,), jnp.int32),
        pltpu.VMEM((GB * RPAD,), jnp.float32),
    ],
)


TB = 64


def _tc_dense(gx_ref, gy_ref, wrows_ref, w1_ref, w2_ref, out_ref):
    gx = gx_ref[...]
    gy = gy_ref[...]
    wx = wrows_ref[0, :]
    wy = wrows_ref[1, :]
    bc = wrows_ref[2, :]
    w1b = wrows_ref[3, :]
    w2b = wrows_ref[4, :]
    gxt = gx.T
    gyt = gy.T
    emb = gxt[:, :, None] * wx[None, None, :] \
        + gyt[:, :, None] * wy[None, None, :] + bc[None, None, :]
    f = jnp.dot(emb[0], w1_ref[...],
                preferred_element_type=jnp.float32) + w1b[None, :]
    l = jnp.dot(emb[1], w2_ref[...],
                preferred_element_type=jnp.float32) + w2b[None, :]
    out_ref[...] = jnp.concatenate(
        [f[None, :, :], l[None, :, :], emb[2:R]], axis=0)


def kernel(nodes, first_node_idx, last_node_idx, candidate_indices,
           W_coord, b_coord, W1_w, W1_b, W2_w, W2_b):
    nodes_t = jnp.transpose(nodes, (1, 2, 0))
    idx = jnp.concatenate(
        [first_node_idx[:, None], last_node_idx[:, None],
         jnp.clip(candidate_indices, 0, None)], axis=1).astype(jnp.int32)
    idx = jnp.pad(idx, ((0, 0), (0, RPAD - R)))
    wrows = jnp.zeros((8, D), jnp.float32)
    wrows = wrows.at[0].set(W_coord[0]).at[1].set(W_coord[1])
    wrows = wrows.at[2].set(b_coord).at[3].set(W1_b).at[4].set(W2_b)

    gx, gy = _sc_call(nodes_t, idx.reshape(B * RPAD))
    gx = gx.reshape(B, RPAD)
    gy = gy.reshape(B, RPAD)

    grid = (B // TB,)
    out_t = pl.pallas_call(
        _tc_dense,
        grid=grid,
        in_specs=[
            pl.BlockSpec((TB, RPAD), lambda i: (i, 0)),
            pl.BlockSpec((TB, RPAD), lambda i: (i, 0)),
            pl.BlockSpec((8, D), lambda i: (0, 0)),
            pl.BlockSpec((D, D), lambda i: (0, 0)),
            pl.BlockSpec((D, D), lambda i: (0, 0)),
        ],
        out_specs=pl.BlockSpec((R, TB, D), lambda i: (0, i, 0)),
        out_shape=jax.ShapeDtypeStruct((R, B, D), jnp.float32),
    )(gx, gy, wrows, W1_w, W2_w)
    return jnp.transpose(out_t, (1, 0, 2))


# speedup vs baseline: 8.2315x; 1.0538x over previous
import jax
import jax.numpy as jnp
from jax import lax
from jax.experimental import pallas as pl
from jax.experimental.pallas import tpu as pltpu
from jax.experimental.pallas import tpu_sc as plsc

B, N, K, D = 1024, 500, 50, 128
R = 2 + K
RPAD = 64
L = 16
NT = 16
GB = 128


def _sc_gather(nodes_hbm, idx_hbm, gx_hbm, gy_hbm, nodes_v, idx_v, g_v):
    wid = lax.axis_index("s") * 2 + lax.axis_index("c")

    @pl.when(wid < NT)
    def _():
        p = wid % 2
        b0 = (wid // 2) * GB
        pltpu.sync_copy(nodes_hbm.at[:, p, pl.ds(b0, GB)], nodes_v)
        pltpu.sync_copy(idx_hbm.at[pl.ds(b0 * RPAD, GB * RPAD)], idx_v)

        jbase = lax.iota(jnp.int32, L)

        def body(b, _):
            bvec = jnp.full((L,), b, jnp.int32)
            for c in range(RPAD // L):
                ids = idx_v[pl.ds(b * RPAD + c * L, L)]
                vals = plsc.load_gather(nodes_v, [ids, bvec])
                plsc.store_scatter(g_v, [jbase + c * L, bvec], vals)
            return 0

        lax.fori_loop(0, GB, body, 0)

        @pl.when(p == 0)
        def _():
            pltpu.sync_copy(g_v, gx_hbm.at[:, pl.ds(b0, GB)])

        @pl.when(p == 1)
        def _():
            pltpu.sync_copy(g_v, gy_hbm.at[:, pl.ds(b0, GB)])


_sc_call = pl.kernel(
    _sc_gather,
    out_type=(
        jax.ShapeDtypeStruct((RPAD, B), jnp.float32),
        jax.ShapeDtypeStruct((RPAD, B), jnp.float32),
    ),
    mesh=plsc.VectorSubcoreMesh(core_axis_name="c", subcore_axis_name="s"),
    compiler_params=pltpu.CompilerParams(needs_layout_passes=False),
    scratch_types=[
        pltpu.VMEM((N, GB), jnp.float32),
        pltpu.VMEM((GB * RPAD,), jnp.int32),
        pltpu.VMEM((RPAD, GB), jnp.float32),
    ],
)


TB = 128


def _tc_dense(gx_ref, gy_ref, wrows_ref, w1_ref, w2_ref, out_ref):
    gxt = gx_ref[...]
    gyt = gy_ref[...]
    wx = wrows_ref[0, :]
    wy = wrows_ref[1, :]
    bc = wrows_ref[2, :]
    w1b = wrows_ref[3, :]
    w2b = wrows_ref[4, :]
    emb = gxt[:, :, None] * wx[None, None, :] \
        + gyt[:, :, None] * wy[None, None, :] + bc[None, None, :]
    f = jnp.dot(emb[0], w1_ref[...],
                preferred_element_type=jnp.float32) + w1b[None, :]
    l = jnp.dot(emb[1], w2_ref[...],
                preferred_element_type=jnp.float32) + w2b[None, :]
    out_ref[...] = jnp.concatenate(
        [f[None, :, :], l[None, :, :], emb[2:R]], axis=0)


def kernel(nodes, first_node_idx, last_node_idx, candidate_indices,
           W_coord, b_coord, W1_w, W1_b, W2_w, W2_b):
    nodes_t = jnp.transpose(nodes, (1, 2, 0))
    idx = jnp.concatenate(
        [first_node_idx[:, None], last_node_idx[:, None],
         jnp.clip(candidate_indices, 0, None)], axis=1).astype(jnp.int32)
    idx = jnp.pad(idx, ((0, 0), (0, RPAD - R)))
    wrows = jnp.concatenate(
        [W_coord, b_coord[None], W1_b[None], W2_b[None],
         jnp.zeros((3, D), jnp.float32)], axis=0)

    gx, gy = _sc_call(nodes_t, idx.reshape(B * RPAD))

    grid = (B // TB,)
    out_t = pl.pallas_call(
        _tc_dense,
        grid=grid,
        in_specs=[
            pl.BlockSpec((RPAD, TB), lambda i: (0, i)),
            pl.BlockSpec((RPAD, TB), lambda i: (0, i)),
            pl.BlockSpec((8, D), lambda i: (0, 0)),
            pl.BlockSpec((D, D), lambda i: (0, 0)),
            pl.BlockSpec((D, D), lambda i: (0, 0)),
        ],
        out_specs=pl.BlockSpec((R, TB, D), lambda i: (0, i, 0)),
        out_shape=jax.ShapeDtypeStruct((R, B, D), jnp.float32),
    )(gx, gy, wrows, W1_w, W2_w)
    return jnp.transpose(out_t, (1, 0, 2))


# speedup vs baseline: 8.4496x; 1.0265x over previous
import functools
import jax
import jax.numpy as jnp
from jax import lax
from jax.experimental import pallas as pl
from jax.experimental.pallas import tpu as pltpu
from jax.experimental.pallas import tpu_sc as plsc

B, N, K, D = 1024, 500, 50, 128
R = 2 + K
RPAD = 64
L = 16
HB = B // 2
GB = 128
NH = 250


def _sc_gather(half, nodes_hbm, idx_hbm, gxa_hbm, gya_hbm, gxb_hbm, gyb_hbm,
               nodes_v, idx_v, g_v):
    wid = lax.axis_index("s") * 2 + lax.axis_index("c")

    @pl.when(wid < 16)
    def _():
        p = wid % 2
        g = (wid // 2) % 4
        nh = wid // 8
        bl = g * GB
        b0 = half * HB + bl
        i0 = nh * NH
        pltpu.sync_copy(nodes_hbm.at[pl.ds(i0, NH), p, pl.ds(b0, GB)],
                        nodes_v)
        pltpu.sync_copy(idx_hbm.at[pl.ds(b0 * RPAD, GB * RPAD)], idx_v)
        jbase = lax.iota(jnp.int32, L)

        @plsc.parallel_loop(0, GB)
        def _gather(b):
            bvec = jnp.full((L,), b, jnp.int32)
            for c in range(RPAD // L):
                ids = idx_v[pl.ds(b * RPAD + c * L, L)] - i0
                valid = (ids >= 0) & (ids < NH)
                safe = jnp.clip(ids, 0, NH - 1)
                vals = plsc.load_gather(nodes_v, [safe, bvec])
                vals = jnp.where(valid, vals, 0.0)
                plsc.store_scatter(g_v, [jbase + c * L, bvec], vals)

        @pl.when((p == 0) & (nh == 0))
        def _():
            pltpu.sync_copy(g_v, gxa_hbm.at[:, pl.ds(bl, GB)])

        @pl.when((p == 1) & (nh == 0))
        def _():
            pltpu.sync_copy(g_v, gya_hbm.at[:, pl.ds(bl, GB)])

        @pl.when((p == 0) & (nh == 1))
        def _():
            pltpu.sync_copy(g_v, gxb_hbm.at[:, pl.ds(bl, GB)])

        @pl.when((p == 1) & (nh == 1))
        def _():
            pltpu.sync_copy(g_v, gyb_hbm.at[:, pl.ds(bl, GB)])


def _make_sc_call(half):
    return pl.kernel(
        functools.partial(_sc_gather, half),
        out_type=tuple(
            jax.ShapeDtypeStruct((RPAD, HB), jnp.float32) for _ in range(4)),
        mesh=plsc.VectorSubcoreMesh(core_axis_name="c", subcore_axis_name="s"),
        compiler_params=pltpu.CompilerParams(needs_layout_passes=False),
        scratch_types=[
            pltpu.VMEM((NH, GB), jnp.float32),
            pltpu.VMEM((GB * RPAD,), jnp.int32),
            pltpu.VMEM((RPAD, GB), jnp.float32),
        ],
    )


_sc_calls = (_make_sc_call(0), _make_sc_call(1))


TB = 128
NSTEP = HB // TB


def _tc_body(gxa_ref, gya_ref, gxb_ref, gyb_ref,
             wrows_ref, w1_ref, w2_ref, out_ref):
    gxt = gxa_ref[...] + gxb_ref[...]
    gyt = gya_ref[...] + gyb_ref[...]
    wx = wrows_ref[0, :]
    wy = wrows_ref[1, :]
    bc = wrows_ref[2, :]
    w1b = wrows_ref[3, :]
    w2b = wrows_ref[4, :]
    emb = gxt[:, :, None] * wx[None, None, :] \
        + gyt[:, :, None] * wy[None, None, :] + bc[None, None, :]
    f = jnp.dot(emb[0], w1_ref[...],
                preferred_element_type=jnp.float32) + w1b[None, :]
    l = jnp.dot(emb[1], w2_ref[...],
                preferred_element_type=jnp.float32) + w2b[None, :]
    out_ref[...] = jnp.concatenate(
        [f[None, :, :], l[None, :, :], emb[2:R]], axis=0)


def _tc_dense0(gxa_ref, gya_ref, gxb_ref, gyb_ref,
               wrows_ref, w1_ref, w2_ref, out_ref):
    _tc_body(gxa_ref, gya_ref, gxb_ref, gyb_ref,
             wrows_ref, w1_ref, w2_ref, out_ref)


def _tc_dense1(buf_ref, gxa_ref, gya_ref, gxb_ref, gyb_ref,
               wrows_ref, w1_ref, w2_ref, out_ref):
    _tc_body(gxa_ref, gya_ref, gxb_ref, gyb_ref,
             wrows_ref, w1_ref, w2_ref, out_ref)


_common_in_specs = [
    pl.BlockSpec((RPAD, TB), lambda i: (0, i)),
    pl.BlockSpec((RPAD, TB), lambda i: (0, i)),
    pl.BlockSpec((RPAD, TB), lambda i: (0, i)),
    pl.BlockSpec((RPAD, TB), lambda i: (0, i)),
    pl.BlockSpec((8, D), lambda i: (0, 0)),
    pl.BlockSpec((D, D), lambda i: (0, 0)),
    pl.BlockSpec((D, D), lambda i: (0, 0)),
]


def kernel(nodes, first_node_idx, last_node_idx, candidate_indices,
           W_coord, b_coord, W1_w, W1_b, W2_w, W2_b):
    nodes_t = jnp.transpose(nodes, (1, 2, 0))
    idx = jnp.concatenate(
        [first_node_idx[:, None], last_node_idx[:, None],
         jnp.clip(candidate_indices, 0, None)], axis=1).astype(jnp.int32)
    idx = jnp.pad(idx, ((0, 0), (0, RPAD - R))).reshape(B * RPAD)
    wrows = jnp.concatenate(
        [W_coord, b_coord[None], W1_b[None], W2_b[None],
         jnp.zeros((3, D), jnp.float32)], axis=0)

    g0 = _sc_calls[0](nodes_t, idx)
    out_half0 = pl.pallas_call(
        _tc_dense0,
        grid=(NSTEP,),
        in_specs=_common_in_specs,
        out_specs=pl.BlockSpec((R, TB, D), lambda i: (0, i, 0)),
        out_shape=jax.ShapeDtypeStruct((R, B, D), jnp.float32),
    )(*g0, wrows, W1_w, W2_w)

    g1 = _sc_calls[1](nodes_t, idx)
    out_t = pl.pallas_call(
        _tc_dense1,
        grid=(NSTEP,),
        in_specs=[pl.BlockSpec(memory_space=pl.ANY)] + _common_in_specs,
        out_specs=pl.BlockSpec((R, TB, D), lambda i: (0, i + NSTEP, 0)),
        out_shape=jax.ShapeDtypeStruct((R, B, D), jnp.float32),
        input_output_aliases={0: 0},
    )(out_half0, *g1, wrows, W1_w, W2_w)

    return jnp.transpose(out_t, (1, 0, 2))
